# baseline (speedup 1.0000x reference)
import functools

import jax
import jax.numpy as jnp
from jax import lax
from jax.experimental import pallas as pl
from jax.experimental.pallas import tpu as pltpu
from jax.experimental.pallas import tpu_sc as plsc

B = 16384
L = 200
E = 32
V = 257

NC = 2
NS = 16
NW = NC * NS
P = B * L
NEG = 4
NUG = NW // NEG
EPG = E // NEG
UW = 1024
UNITS = P // UW
UPG = UNITS // NUG
SU = 2
SW = SU * UW
NIT = UPG // SU
NV = SW // 16
NBUF = 4


def _emb_body(tab_hbm, x_hbm, out_hbm, tab_v,
              x_bufs, out_bufs, isems, osems):
    cid = lax.axis_index("c")
    sid = lax.axis_index("s")
    wid = sid * NC + cid
    eg = lax.rem(wid, NEG)
    ug = lax.div(wid, NEG)
    e0 = eg * EPG

    pltpu.sync_copy(tab_hbm.at[pl.ds(e0 * V * 16, EPG * V * 16)], tab_v)
    lanes = lax.broadcasted_iota(jnp.int32, (16,), 0)

    def x_copy(i, v, sem):
        off = (ug * UPG + i * SU) * UW
        return pltpu.make_async_copy(
            x_hbm.at[pl.ds(off, SW)], v, sem)

    def out_copies(i, v, sem):
        u0 = ug * UPG + i * SU
        return [pltpu.make_async_copy(
                    v.at[pl.ds(j * SW, SW)],
                    out_hbm.at[pl.ds((e0 + j) * P + u0 * UW, SW)], sem)
                for j in range(EPG)]

    def out_start(i, v, sem):
        for c in out_copies(i, v, sem):
            c.start()

    def out_wait(i, v, sem):
        for c in out_copies(i, v, sem):
            c.wait()

    def compute(xv, ov):
        def vec(vi, _):
            iv = xv[pl.ds(vi * 16, 16)] * 16 + lanes
            gs = [plsc.load_gather(tab_v, [iv + j * (V * 16)])
                  for j in range(EPG)]
            for j in range(EPG):
                ov[pl.ds(j * SW + vi * 16, 16)] = gs[j]
            return 0
        lax.fori_loop(0, NV, vec, 0, unroll=16)

    bufs = tuple(zip(x_bufs, out_bufs, isems, osems))

    for i in range(NBUF):
        x_copy(i, x_bufs[i], isems[i]).start()
    for i in range(NBUF):
        xv, ov, isem, osem = bufs[i]
        x_copy(i, xv, isem).wait()
        compute(xv, ov)
        out_start(i, ov, osem)
        x_copy(i + NBUF, xv, isem).start()

    def steady(h, _):
        for b in range(NBUF):
            i = h * NBUF + b
            xv, ov, isem, osem = bufs[b]
            x_copy(i, xv, isem).wait()
            out_wait(i - NBUF, ov, osem)
            compute(xv, ov)
            out_start(i, ov, osem)
            x_copy(lax.rem(i + NBUF, NIT), xv, isem).start()
        return 0
    lax.fori_loop(1, NIT // NBUF, steady, 0)

    for b in range(NBUF):
        xv, ov, isem, osem = bufs[b]
        out_wait(NIT - NBUF + b, ov, osem)
        x_copy(b, xv, isem).wait()


@functools.partial(jax.jit, static_argnames=())
def kernel(input_x, table):
    xs = jnp.squeeze(input_x, 1).astype(jnp.int32)
    x_phys = xs.reshape(128, 128, 25, 8).transpose(2, 0, 3, 1).reshape(-1)
    tab_t = jnp.broadcast_to(jnp.transpose(table).reshape(-1)[:, None],
                             (E * V, 16)).reshape(-1)

    run = pl.kernel(
        _emb_body,
        out_type=jax.ShapeDtypeStruct((E * P,), jnp.float32),
        mesh=plsc.VectorSubcoreMesh(
            core_axis_name="c", subcore_axis_name="s",
            num_cores=NC, num_subcores=NS),
        scratch_types=[
            pltpu.VMEM((EPG * V * 16,), jnp.float32),
            tuple(pltpu.VMEM((SW,), jnp.int32) for _ in range(NBUF)),
            tuple(pltpu.VMEM((EPG * SW,), jnp.float32) for _ in range(NBUF)),
            tuple(pltpu.SemaphoreType.DMA for _ in range(NBUF)),
            tuple(pltpu.SemaphoreType.DMA for _ in range(NBUF)),
        ],
        compiler_params=pltpu.CompilerParams(needs_layout_passes=False),
    )
    out_phys = run(tab_t, x_phys)
    return (out_phys.reshape(E, 25, 128, 8, 128)
            .transpose(2, 4, 0, 1, 3).reshape(B, E, L))


# speedup vs baseline: 1.6247x; 1.6247x over previous
import functools

import jax
import jax.numpy as jnp
from jax import lax
from jax.experimental import pallas as pl
from jax.experimental.pallas import tpu as pltpu
from jax.experimental.pallas import tpu_sc as plsc

B = 16384
L = 200
E = 32
V = 257

NC = 2
NS = 16
NW = NC * NS
P = B * L
NEG = 4
NUG = NW // NEG
EPG = E // NEG
UW = 1024
UNITS = P // UW
UPG = UNITS // NUG
SU = 4
SW = SU * UW
NIT = UPG // SU
NV = SW // 16
NBUF = 2


def _emb_body(tab_hbm, x_hbm, out_hbm, tab_v,
              x_bufs, out_bufs, isems, osems):
    cid = lax.axis_index("c")
    sid = lax.axis_index("s")
    wid = sid * NC + cid
    eg = lax.rem(wid, NEG)
    ug = lax.div(wid, NEG)
    e0 = eg * EPG

    pltpu.sync_copy(tab_hbm.at[pl.ds(e0 * V * 16, EPG * V * 16)], tab_v)
    lanes = lax.broadcasted_iota(jnp.int32, (16,), 0)

    def x_copy(i, v, sem):
        off = (ug * UPG + i * SU) * UW
        return pltpu.make_async_copy(
            x_hbm.at[pl.ds(off, SW)], v, sem)

    def out_copies(i, v, sem):
        u0 = ug * UPG + i * SU
        return [pltpu.make_async_copy(
                    v.at[pl.ds(j * SW, SW)],
                    out_hbm.at[pl.ds((e0 + j) * P + u0 * UW, SW)], sem)
                for j in range(EPG)]

    def out_start(i, v, sem):
        for c in out_copies(i, v, sem):
            c.start()

    def out_wait(i, v, sem):
        for c in out_copies(i, v, sem):
            c.wait()

    def compute(xv, ov):
        def vec(vi, _):
            iv = xv[pl.ds(vi * 16, 16)] * 16 + lanes
            gs = [plsc.load_gather(tab_v, [iv + j * (V * 16)])
                  for j in range(EPG)]
            for j in range(EPG):
                ov[pl.ds(j * SW + vi * 16, 16)] = gs[j]
            return 0
        lax.fori_loop(0, NV, vec, 0, unroll=16)

    bufs = tuple(zip(x_bufs, out_bufs, isems, osems))

    for i in range(NBUF):
        x_copy(i, x_bufs[i], isems[i]).start()
    for i in range(NBUF):
        xv, ov, isem, osem = bufs[i]
        x_copy(i, xv, isem).wait()
        compute(xv, ov)
        out_start(i, ov, osem)
        x_copy(i + NBUF, xv, isem).start()

    def steady(h, _):
        for b in range(NBUF):
            i = h * NBUF + b
            xv, ov, isem, osem = bufs[b]
            x_copy(i, xv, isem).wait()
            out_wait(i - NBUF, ov, osem)
            compute(xv, ov)
            out_start(i, ov, osem)
            x_copy(lax.rem(i + NBUF, NIT), xv, isem).start()
        return 0
    lax.fori_loop(1, NIT // NBUF, steady, 0)

    for b in range(NBUF):
        xv, ov, isem, osem = bufs[b]
        out_wait(NIT - NBUF + b, ov, osem)
        x_copy(b, xv, isem).wait()


@functools.partial(jax.jit, static_argnames=())
def kernel(input_x, table):
    xs = jnp.squeeze(input_x, 1).astype(jnp.int32)
    x_phys = xs.reshape(128, 128, 25, 8).transpose(2, 0, 3, 1).reshape(-1)
    tab_t = jnp.broadcast_to(jnp.transpose(table).reshape(-1)[:, None],
                             (E * V, 16)).reshape(-1)

    run = pl.kernel(
        _emb_body,
        out_type=jax.ShapeDtypeStruct((E * P,), jnp.float32),
        mesh=plsc.VectorSubcoreMesh(
            core_axis_name="c", subcore_axis_name="s",
            num_cores=NC, num_subcores=NS),
        scratch_types=[
            pltpu.VMEM((EPG * V * 16,), jnp.float32),
            tuple(pltpu.VMEM((SW,), jnp.int32) for _ in range(NBUF)),
            tuple(pltpu.VMEM((EPG * SW,), jnp.float32) for _ in range(NBUF)),
            tuple(pltpu.SemaphoreType.DMA for _ in range(NBUF)),
            tuple(pltpu.SemaphoreType.DMA for _ in range(NBUF)),
        ],
        compiler_params=pltpu.CompilerParams(needs_layout_passes=False),
    )
    out_phys = run(tab_t, x_phys)
    return (out_phys.reshape(E, 25, 128, 8, 128)
            .transpose(2, 4, 0, 1, 3).reshape(B, E, L))
